# Initial kernel scaffold; baseline (speedup 1.0000x reference)
import jax
import jax.numpy as jnp
from jax.experimental import pallas as pl


def kernel(positions, table):
    raise NotImplementedError("write your pallas kernel here")


# speedup vs baseline: 2.0890x; 2.0890x over previous
import functools

import jax
import jax.numpy as jnp
from jax import lax
from jax.experimental import pallas as pl
from jax.experimental.pallas import tpu as pltpu
from jax.experimental.pallas import tpu_sc as plsc

_NC = 2
_NS = 16
_NW = _NC * _NS
_CHUNK = 32
_NCHUNK = 16
_BPW = _CHUNK * _NCHUNK
_D = 1024


def _sc_gather(idx_hbm, table_hbm, out_hbm, idx_v, rows_v, gsem, ssem):
    wid = lax.axis_index("s") * _NC + lax.axis_index("c")
    base = wid * _BPW

    pltpu.sync_copy(idx_hbm.at[wid], idx_v)

    scat = [None] * _NCHUNK
    gath = [None] * _NCHUNK
    gath[0] = pltpu.async_copy(table_hbm.at[idx_v.at[0]], rows_v.at[0], gsem)
    for c in range(_NCHUNK):
        buf = c & 1
        if c + 1 < _NCHUNK:
            if c >= 1:
                scat[c - 1].wait()
            gath[c + 1] = pltpu.async_copy(
                table_hbm.at[idx_v.at[c + 1]], rows_v.at[(c + 1) & 1], gsem)
        gath[c].wait()
        scat[c] = pltpu.async_copy(
            rows_v.at[buf], out_hbm.at[pl.ds(base + c * _CHUNK, _CHUNK)], ssem)
    if _NCHUNK >= 2:
        scat[_NCHUNK - 2].wait()
    scat[_NCHUNK - 1].wait()


@jax.jit
def _run(idx, table):
    k = functools.partial(
        pl.kernel,
        mesh=plsc.VectorSubcoreMesh(core_axis_name="c", subcore_axis_name="s"),
        out_type=jax.ShapeDtypeStruct((_NW * _BPW, _D), jnp.float32),
        scratch_types=[
            pltpu.VMEM((_NCHUNK, _CHUNK), jnp.int32),
            pltpu.VMEM((2, _CHUNK, _D), jnp.float32),
            pltpu.SemaphoreType.DMA,
            pltpu.SemaphoreType.DMA,
        ],
    )(_sc_gather)
    return k(idx, table)


def kernel(positions, table):
    b, s = positions.shape
    idx = positions.astype(jnp.int32).reshape(_NW, _NCHUNK, _CHUNK)
    out = _run(idx, table)
    return out.reshape(b, s, _D)
